# baseline (speedup 1.0000x reference)
import functools

import jax
import jax.numpy as jnp
from jax import lax
from jax.experimental import pallas as pl
from jax.experimental.pallas import tpu as pltpu
from jax.experimental.pallas import tpu_sc as plsc

_NC = 2
_NS = 16
_NW = _NC * _NS
_L = 16
_PW = 128
_CAP = 16512
_OCAP = 4096


def _phase1(idx, table_t):
    (b,) = idx.shape
    d, v = table_t.shape
    n_full = v // _PW
    n_pan = n_full + (1 if v % _PW else 0)
    ppt = (n_pan + _NW - 1) // _NW
    tail = v - n_full * _PW
    mesh = plsc.VectorSubcoreMesh(core_axis_name="c", subcore_axis_name="s")
    i32 = jnp.int32

    @functools.partial(
        pl.kernel,
        mesh=mesh,
        out_type=(
            jax.ShapeDtypeStruct((_NW * _CAP * _PW,), jnp.float32),
            jax.ShapeDtypeStruct((_NW * _OCAP,), i32),
            jax.ShapeDtypeStruct((_NW * _L,), i32),
        ),
        compiler_params=pltpu.CompilerParams(needs_layout_passes=False),
        scratch_types=[
            pltpu.VMEM((1024,), i32),
            pltpu.VMEM((_OCAP + _L,), i32),
            pltpu.VMEM((_OCAP + _L,), i32),
            pltpu.VMEM((_OCAP,), i32),
            pltpu.VMEM((2, d, _PW), jnp.float32),
            pltpu.VMEM((d, 64), jnp.float32),
            pltpu.VMEM((_PW,), jnp.float32),
            pltpu.VMEM((2 * _L,), i32),
            pltpu.VMEM((2 * _L,), i32),
            pltpu.VMEM((_L,), i32),
            pltpu.SemaphoreType.DMA,
        ],
    )
    def k(idx_hbm, tbl_hbm, rows, ppos, counts, idxc, oidx, opos, hitp,
          panel, tailb, rowb, hidxb, hposb, cntb, semp):
        c = lax.axis_index("c")
        s = lax.axis_index("s")
        w = s * _NC + c
        iv = lax.iota(i32, _L)
        lo_p = w * ppt
        hi_p = jnp.minimum(lo_p + ppt, n_pan)

        def scan_pass(ps, cnt):
            pltpu.sync_copy(idx_hbm.at[pl.ds(ps * 1024, 1024)], idxc)

            def scan(t, cnt):
                vv = idxc[pl.ds(t * _L, _L)]
                pan = lax.shift_right_logical(vv, 7)
                own = jnp.logical_and(pan >= lo_p, pan < hi_p)
                owni = own.astype(i32)
                pfx = plsc.cumsum(owni)
                slot = jnp.where(
                    own, jnp.minimum(cnt + pfx - 1, _OCAP - 1),
                    _OCAP + iv)
                plsc.store_scatter(oidx, [slot], vv)
                plsc.store_scatter(opos, [slot], ps * 1024 + t * _L + iv)
                return cnt + pfx[_L - 1]

            return lax.fori_loop(0, 1024 // _L, scan, cnt)

        cnt = jnp.minimum(
            lax.fori_loop(0, b // 1024, scan_pass, jnp.int32(0)),
            jnp.int32(_OCAP))
        nck = lax.div(cnt + (_L - 1), jnp.int32(_L))
        cntb[pl.ds(0, _L)] = jnp.full((_L,), cnt, i32)
        pltpu.sync_copy(cntb, counts.at[pl.ds(w * _L, _L)])

        nfull_t = jnp.maximum(jnp.minimum(hi_p, n_full) - lo_p, 0)

        def issue(j, buf):
            po = pl.multiple_of((lo_p + j) * _PW, _PW)
            pltpu.async_copy(
                tbl_hbm.at[:, pl.ds(po, _PW)], panel.at[buf], semp)

        @pl.when(nfull_t > 0)
        def _():
            issue(0, 0)

        def extract_hits(pg, pref, hc):
            def mbody(kk, hc):
                ov = oidx[pl.ds(kk * _L, _L)]
                m = lax.shift_right_logical(ov, 7) == pg
                nh = plsc.all_reduce_population_count(m)[0]
                qv = opos[pl.ds(kk * _L, _L)]
                mi = m.astype(i32)
                hpfx = plsc.cumsum(mi)
                hslot = jnp.where(m, hpfx - 1, _L + iv)
                plsc.store_scatter(hidxb, [hslot], ov)
                plsc.store_scatter(hposb, [hslot], qv)
                def hbody(h, hc):
                    lane = jnp.bitwise_and(hidxb[pl.ds(h, _L)][0], _PW - 1)
                    posn = hposb[pl.ds(h, _L)][0]
                    hcc = jnp.minimum(hc, jnp.int32(_OCAP - 1))
                    plsc.store_scatter(
                        hitp, [jnp.full((_L,), hcc, i32)],
                        jnp.full((_L,), posn, i32))
                    lb = jnp.full((_L,), lane, i32)
                    for dg in range(d // _L):
                        vals = plsc.load_gather(pref, [dg * _L + iv, lb])
                        rowb[pl.ds(dg * _L, _L)] = vals
                    pltpu.sync_copy(
                        rowb, rows.at[pl.ds((w * _CAP + hcc) * _PW, _PW)])
                    return hc + 1

                return lax.fori_loop(0, nh, hbody, hc)

            return lax.fori_loop(0, nck, mbody, hc)

        def pbody(j, hc):
            buf = lax.rem(j, jnp.int32(2))
            pltpu.make_async_copy(
                tbl_hbm.at[:, pl.ds(0, _PW)], panel.at[buf], semp).wait()
            jn = jnp.minimum(j + 1, nfull_t - 1)
            issue(jn, 1 - buf)
            return extract_hits(lo_p + j, panel.at[buf], hc)

        hc = lax.fori_loop(0, nfull_t, pbody, jnp.int32(0))

        def drain(r, _):
            pltpu.make_async_copy(
                tbl_hbm.at[:, pl.ds(0, _PW)], panel.at[0], semp).wait()
            return ()

        lax.fori_loop(0, jnp.minimum(nfull_t, 1), drain, ())

        if tail:
            @pl.when(hi_p == jnp.int32(n_pan))
            def _():
                pltpu.sync_copy(tbl_hbm.at[:, pl.ds(v - tail, tail)], tailb)
                extract_hits(jnp.int32(n_full), tailb, hc)

        pltpu.sync_copy(hitp, ppos.at[pl.ds(w * _OCAP, _OCAP)])

    return k(idx, table_t)


def _phase2(rows2d, ppos, counts, b, d):
    mesh = plsc.VectorSubcoreMesh(core_axis_name="c", subcore_axis_name="s")
    i32 = jnp.int32
    bpw = b // _NW

    @functools.partial(
        pl.kernel,
        mesh=mesh,
        out_type=jax.ShapeDtypeStruct((b, d), jnp.float32),
        compiler_params=pltpu.CompilerParams(needs_layout_passes=False),
        scratch_types=[
            pltpu.VMEM((_NW * _L,), i32),
            pltpu.VMEM((bpw + _L,), i32),
            pltpu.VMEM((256,), i32),
            pltpu.VMEM((_PW, _PW), jnp.float32),
            pltpu.VMEM((_PW, d), jnp.float32),
            pltpu.SemaphoreType.DMA,
        ],
    )
    def k(rows_hbm, ppos_hbm, counts_hbm, out_hbm, cnts, smap,
          pbp, gbuf, obuf, sem):
        c = lax.axis_index("c")
        s = lax.axis_index("s")
        w = s * _NC + c
        iv = lax.iota(i32, _L)
        mylo = w * bpw
        pltpu.sync_copy(counts_hbm, cnts)

        for r in range(_NW):
            cnt_r = cnts[pl.ds(r * _L, _L)][0]
            n2 = lax.div(cnt_r + 255, jnp.int32(256))

            def outer(k2, _):
                off = r * _OCAP + k2 * 256
                pltpu.sync_copy(ppos_hbm.at[pl.ds(off, 256)], pbp)
                nin = jnp.minimum(cnt_r - k2 * 256, 256)

                def inner(t, _):
                    pv = pbp[pl.ds(t * _L, _L)]
                    sv = r * _CAP + k2 * 256 + t * _L + iv
                    valid = (t * _L + iv) < nin
                    inr = jnp.logical_and(pv >= mylo, pv < mylo + bpw)
                    keep = jnp.logical_and(valid, inr)
                    tgt = jnp.where(keep, pv - mylo, bpw + iv)
                    plsc.store_scatter(smap, [tgt], sv)
                    return ()

                lax.fori_loop(0, 256 // _L, inner, ())
                return ()

            lax.fori_loop(0, n2, outer, ())

        for ch in range(bpw // _PW):
            pltpu.async_copy(
                rows_hbm.at[smap.at[pl.ds(ch * _PW, _PW)]], gbuf, sem)
            pltpu.make_async_copy(
                rows_hbm.at[pl.ds(0, _PW)], gbuf, sem).wait()

            def ext(i, _):
                for cc in range(d // _L):
                    obuf[i, pl.ds(cc * _L, _L)] = gbuf[i, pl.ds(cc * _L, _L)]
                return ()

            lax.fori_loop(0, _PW, ext, ())
            pltpu.sync_copy(
                obuf, out_hbm.at[pl.ds(mylo + ch * _PW, _PW)])

    return k(rows2d, ppos, counts)


def kernel(input, use_blank, has_blank, table):
    idx = input.astype(jnp.int32)
    rows1d, ppos, counts = _phase1(idx, table.T)
    rows2d = rows1d.reshape(_NW * _CAP, _PW)
    b = idx.shape[0]
    return _phase2(rows2d, ppos, counts, b, table.shape[1])
